# baseline (speedup 1.0000x reference)
import jax
import jax.numpy as jnp
from jax import lax
from jax.experimental import pallas as pl
from jax.experimental.pallas import tpu as pltpu
from jax.experimental.pallas import tpu_sc as plsc

N = 10000
E = 320000
D = 128
DH = D // 2
D_OUT = 40

NC = 2
NCU = 2
NS = 16
NW = NCU * NS

CE = 128
KP = 40
NPH0 = 2
NPH1 = 2
K0 = NPH0 * KP
K1 = NPH1 * KP
E_PAD = NS * (K0 + K1) * CE
NP = 10240
RPT = NP // NS

BM = 256


def _mesh():
    return plsc.VectorSubcoreMesh(core_axis_name="c", subcore_axis_name="s", num_cores=NCU)


def _degree_body(src_hbm, dst_hbm, degp_hbm, sidx, didx, hist_s, hist_d):
    c = lax.axis_index("c")
    s = lax.axis_index("s")
    wid = c * NS + s

    pltpu.sync_copy(src_hbm.at[wid], sidx)
    pltpu.sync_copy(dst_hbm.at[wid], didx)

    zero16 = jnp.zeros((16,), jnp.float32)

    def zinit(i, _):
        hist_s[pl.ds(i * 16, 16)] = zero16
        hist_d[pl.ds(i * 16, 16)] = zero16
        return 0

    lax.fori_loop(0, NP // 16, zinit, 0)

    ones16 = jnp.ones((16,), jnp.float32)

    def body(i, _):
        sv = sidx[pl.ds(i * 16, 16)]
        dv = didx[pl.ds(i * 16, 16)]
        plsc.addupdate_scatter(hist_s, [sv], ones16)
        plsc.addupdate_scatter(hist_d, [dv], ones16)
        return 0

    lax.fori_loop(0, (E_PAD // NW) // 16, body, 0)

    pltpu.sync_copy(hist_s, degp_hbm.at[0, wid])
    pltpu.sync_copy(hist_d, degp_hbm.at[1, wid])


@jax.jit
def _degrees(src_p, dst_p):
    return pl.kernel(
        _degree_body,
        out_type=jax.ShapeDtypeStruct((2, NW, NP), jnp.float32),
        mesh=_mesh(),
        compiler_params=pltpu.CompilerParams(needs_layout_passes=False),
        scratch_types=[
            pltpu.VMEM((E_PAD // NW,), jnp.int32),
            pltpu.VMEM((E_PAD // NW,), jnp.int32),
            pltpu.VMEM((NP,), jnp.float32),
            pltpu.VMEM((NP,), jnp.float32),
        ],
    )(src_p, dst_p)


def _agg_body(y_hbm, edge_hbm, zeros_hbm, out_hbm, eidx, rb0, rb1, rf, acc,
              gsem):
    c = lax.axis_index("c")
    s = lax.axis_index("s")
    wid = c * NS + s
    rb = [rb0, rb1]

    pltpu.sync_copy(zeros_hbm.at[pl.ds(s * RPT, RPT)],
                    acc.at[pl.ds(s * RPT, RPT)])

    plsc.subcore_barrier()

    mhi = jnp.full((16,), -65536, jnp.int32)

    def convert(bsrc):
        def row(i, _):
            for k in range(4):
                w = bsrc[i, pl.ds(k * 16, 16)]
                lo = plsc.bitcast(w << 16, jnp.float32)
                hi = plsc.bitcast(w & mhi, jnp.float32)
                rf[i, pl.ds(k * 16, 16)] = lo
                rf[i, pl.ds(64 + k * 16, 16)] = hi
            return 0

        lax.fori_loop(0, CE, row, 0)

    def phase_body(ph, _):
        pltpu.sync_copy(edge_hbm.at[wid, ph], eidx)

        for b in range(2):
            pltpu.async_copy(y_hbm.at[eidx.at[0, b]], rb[b], gsem.at[b])

        def round_body(r, _):
            for b in range(2):
                j = 2 * r + b
                pltpu.make_async_copy(
                    y_hbm.at[eidx.at[0, j]], rb[b], gsem.at[b]).wait()
                convert(rb[b])
                pltpu.async_copy(y_hbm.at[eidx.at[0, j + 2]], rb[b],
                                 gsem.at[b])
                pltpu.sync_copy(rf, acc.at[eidx.at[1, j]], add=True)
            return 0

        lax.fori_loop(0, KP // 2 - 1, round_body, 0)
        for b in range(2):
            j = KP - 2 + b
            pltpu.make_async_copy(
                y_hbm.at[eidx.at[0, j]], rb[b], gsem.at[b]).wait()
            convert(rb[b])
            pltpu.sync_copy(rf, acc.at[eidx.at[1, j]], add=True)
        return 0

    lax.fori_loop(0, NPH0, phase_body, 0)

    plsc.subcore_barrier()

    pltpu.sync_copy(acc.at[pl.ds(s * RPT, RPT)],
                    out_hbm.at[c, pl.ds(s * RPT, RPT)])


@jax.jit
def _aggregate(y_p, edge_p, zeros_p):
    return pl.kernel(
        _agg_body,
        out_type=jax.ShapeDtypeStruct((NCU, NP, D), jnp.float32),
        mesh=_mesh(),
        compiler_params=pltpu.CompilerParams(needs_layout_passes=False,
                                             use_tc_tiling_on_sc=False),
        scratch_types=[
            pltpu.VMEM((2, KP, CE), jnp.int32),
            pltpu.VMEM((CE, DH), jnp.int32),
            pltpu.VMEM((CE, DH), jnp.int32),
            pltpu.VMEM((CE, D), jnp.float32),
            pltpu.VMEM_SHARED((NP, D), jnp.float32),
            pltpu.SemaphoreType.DMA((2,)),
        ],
    )(y_p, edge_p, zeros_p)


def _norms(degb):
    deg_src = jnp.sum(degb[:NW], axis=0)
    deg_dst = jnp.sum(degb[NW:], axis=0)
    n_src = lax.rsqrt(jnp.clip(deg_src, 1.0, None))
    n_dst = lax.rsqrt(jnp.clip(deg_dst, 1.0, None))
    return n_src, n_dst


def _pack_y(y):
    return jnp.stack([y[:, :DH], y[:, DH:]],
                     axis=-1).reshape(y.shape[0], D).astype(jnp.bfloat16)


def _tca_body(xb, w1, degb, yb):
    n_src, _ = _norms(degb)
    y = jnp.dot(xb[...], w1[...],
                preferred_element_type=jnp.float32) * n_src[:, None]
    yb[...] = _pack_y(y)


@jax.jit
def _tc_a(x_p, W1, degp):
    grid = NP // BM
    return pl.pallas_call(
        _tca_body,
        grid=(grid,),
        in_specs=[
            pl.BlockSpec((BM, D), lambda i: (i, 0)),
            pl.BlockSpec((D, D), lambda i: (0, 0)),
            pl.BlockSpec((2 * NW, BM), lambda i: (0, i)),
        ],
        out_specs=pl.BlockSpec((BM, D), lambda i: (i, 0)),
        out_shape=jax.ShapeDtypeStruct((NP, D), jnp.bfloat16),
    )(x_p, W1, degp)


def _tcb_body(aggb, w2, b1b, degb, yb):
    n_src, n_dst = _norms(degb)
    row = pl.program_id(0) * BM + lax.broadcasted_iota(jnp.int32, (BM,), 0)
    n_src = jnp.where(row < N, n_src, 0.0)
    agg = sum(aggb[i] for i in range(1, NCU)) + aggb[0]
    h = jax.nn.relu(agg * n_dst[:, None] + b1b[...])
    y = jnp.dot(h, w2[...],
                preferred_element_type=jnp.float32) * n_src[:, None]
    yb[...] = _pack_y(y)


@jax.jit
def _tc_b(agg, W2, b1, degp):
    grid = NP // BM
    return pl.pallas_call(
        _tcb_body,
        grid=(grid,),
        in_specs=[
            pl.BlockSpec((NCU, BM, D), lambda i: (0, i, 0)),
            pl.BlockSpec((D, D), lambda i: (0, 0)),
            pl.BlockSpec((1, D), lambda i: (0, 0)),
            pl.BlockSpec((2 * NW, BM), lambda i: (0, i)),
        ],
        out_specs=pl.BlockSpec((BM, D), lambda i: (i, 0)),
        out_shape=jax.ShapeDtypeStruct((NP, D), jnp.bfloat16),
    )(agg, W2, b1.reshape(1, D), degp)


def _tcc_body(aggb, wc, b2b, bcb, degb, ob):
    _, n_dst = _norms(degb)
    agg = sum(aggb[i] for i in range(1, NCU)) + aggb[0]
    h = jax.nn.relu(agg * n_dst[:, None] + b2b[...])
    ob[...] = jnp.dot(h, wc[...], preferred_element_type=jnp.float32) + bcb[...]


@jax.jit
def _tc_c(agg, Wc_p, b2, bc_p, degp):
    grid = NP // BM
    return pl.pallas_call(
        _tcc_body,
        grid=(grid,),
        in_specs=[
            pl.BlockSpec((NCU, BM, D), lambda i: (0, i, 0)),
            pl.BlockSpec((D, D), lambda i: (0, 0)),
            pl.BlockSpec((1, D), lambda i: (0, 0)),
            pl.BlockSpec((1, D), lambda i: (0, 0)),
            pl.BlockSpec((2 * NW, BM), lambda i: (0, i)),
        ],
        out_specs=pl.BlockSpec((BM, D), lambda i: (i, 0)),
        out_shape=jax.ShapeDtypeStruct((NP, D), jnp.float32),
    )(agg, Wc_p, b2.reshape(1, D), bc_p, degp)


def kernel(x, edge_index, W1, b1, W2, b2, Wc, bc):
    src = edge_index[0].astype(jnp.int32)
    dst = edge_index[1].astype(jnp.int32)
    padfill = jnp.full((E_PAD - E,), N, jnp.int32)
    src_p = jnp.concatenate([src, padfill])
    dst_p = jnp.concatenate([dst, padfill])

    e0 = NS * K0 * CE

    def _tile_view(a):
        a0 = a[:e0].reshape(NS, NPH0, KP, CE)
        a1 = a[e0:].reshape(NS, NPH1, KP, CE)
        a1 = jnp.pad(a1, ((0, 0), (0, NPH0 - NPH1), (0, 0), (0, 0)),
                     constant_values=N)
        return jnp.concatenate([a1, a0], axis=0)

    edge_p = jnp.stack([_tile_view(src_p), _tile_view(dst_p)], axis=2)
    zeros_p = jnp.zeros((NP, D), jnp.float32)
    src_d = src_p.reshape(NW, E_PAD // NW)
    dst_d = dst_p.reshape(NW, E_PAD // NW)

    x_p = jnp.pad(x, ((0, NP - N), (0, 0)))
    Wc_p = jnp.pad(Wc, ((0, 0), (0, D - D_OUT)))
    bc_p = jnp.pad(bc, ((0, D - D_OUT),)).reshape(1, D)

    degp = _degrees(src_d, dst_d).reshape(2 * NW, NP)

    y1 = _tc_a(x_p, W1, degp)
    y1v = lax.bitcast_convert_type(y1.reshape(NP, DH, 2), jnp.int32)
    agg1 = _aggregate(y1v, edge_p, zeros_p)
    y2 = _tc_b(agg1, W2, b1, degp)
    y2v = lax.bitcast_convert_type(y2.reshape(NP, DH, 2), jnp.int32)
    agg2 = _aggregate(y2v, edge_p, zeros_p)
    out = _tc_c(agg2, Wc_p, b2, bc_p, degp)
    return out[:N, :D_OUT]


# speedup vs baseline: 1.3842x; 1.3842x over previous
import jax
import jax.numpy as jnp
from jax import lax
from jax.experimental import pallas as pl
from jax.experimental.pallas import tpu as pltpu
from jax.experimental.pallas import tpu_sc as plsc

N = 10000
E = 320000
D = 128
DH = D // 2
D_OUT = 40

NC = 2
NCU = 2
NS = 16
NW = NCU * NS

CE = 128
KP = 40
NPH0 = 2
NPH1 = 2
K0 = NPH0 * KP
K1 = NPH1 * KP
E_PAD = NS * (K0 + K1) * CE
NP = 10240
RPT = NP // NS

BM = 256


def _mesh():
    return plsc.VectorSubcoreMesh(core_axis_name="c", subcore_axis_name="s", num_cores=NCU)


def _degree_body(src_hbm, dst_hbm, degp_hbm, sidx, didx, hist_s, hist_d):
    c = lax.axis_index("c")
    s = lax.axis_index("s")
    wid = c * NS + s

    pltpu.sync_copy(src_hbm.at[wid], sidx)
    pltpu.sync_copy(dst_hbm.at[wid], didx)

    zero16 = jnp.zeros((16,), jnp.float32)

    def zinit(i, _):
        hist_s[pl.ds(i * 16, 16)] = zero16
        hist_d[pl.ds(i * 16, 16)] = zero16
        return 0

    lax.fori_loop(0, NP // 16, zinit, 0)

    ones16 = jnp.ones((16,), jnp.float32)

    def body(i, _):
        sv = sidx[pl.ds(i * 16, 16)]
        dv = didx[pl.ds(i * 16, 16)]
        plsc.addupdate_scatter(hist_s, [sv], ones16)
        plsc.addupdate_scatter(hist_d, [dv], ones16)
        return 0

    lax.fori_loop(0, (E_PAD // NW) // 16, body, 0)

    pltpu.sync_copy(hist_s, degp_hbm.at[0, wid])
    pltpu.sync_copy(hist_d, degp_hbm.at[1, wid])


@jax.jit
def _degrees(src_p, dst_p):
    return pl.kernel(
        _degree_body,
        out_type=jax.ShapeDtypeStruct((2, NW, NP), jnp.float32),
        mesh=_mesh(),
        compiler_params=pltpu.CompilerParams(needs_layout_passes=False),
        scratch_types=[
            pltpu.VMEM((E_PAD // NW,), jnp.int32),
            pltpu.VMEM((E_PAD // NW,), jnp.int32),
            pltpu.VMEM((NP,), jnp.float32),
            pltpu.VMEM((NP,), jnp.float32),
        ],
    )(src_p, dst_p)


def _agg_body(y_hbm, edge_hbm, zeros_hbm, out_hbm, eidx, rr, acc, gsem):
    c = lax.axis_index("c")
    s = lax.axis_index("s")
    wid = c * NS + s

    pltpu.sync_copy(zeros_hbm.at[pl.ds(s * RPT, RPT)],
                    acc.at[pl.ds(s * RPT, RPT)])

    plsc.subcore_barrier()

    nph = NPH0

    HCE = CE // 2

    def gather(j, b):
        for h in range(2):
            pltpu.async_copy(
                y_hbm.at[eidx.at[0, j, pl.ds(h * HCE, HCE)]],
                rr.at[b, pl.ds(h * HCE, HCE)], gsem.at[2 * b + h])

    def gather_wait(j, b):
        for h in range(2):
            pltpu.make_async_copy(
                y_hbm.at[eidx.at[0, j, pl.ds(h * HCE, HCE)]],
                rr.at[b, pl.ds(h * HCE, HCE)], gsem.at[2 * b + h]).wait()

    def phase_body(ph, _):
        pltpu.sync_copy(edge_hbm.at[wid, ph], eidx)

        for b in range(2):
            gather(b, b)

        def round_body(r, _):
            for b in range(2):
                j = 2 * r + b
                gather_wait(j, b)
                pltpu.sync_copy(rr.at[b], acc.at[eidx.at[1, j]], add=True)
                gather(j + 2, b)
            return 0

        lax.fori_loop(0, KP // 2 - 1, round_body, 0)
        for b in range(2):
            j = KP - 2 + b
            gather_wait(j, b)
            pltpu.sync_copy(rr.at[b], acc.at[eidx.at[1, j]], add=True)
        return 0

    lax.fori_loop(0, nph, phase_body, 0)

    plsc.subcore_barrier()

    pltpu.sync_copy(acc.at[pl.ds(s * RPT, RPT)],
                    out_hbm.at[c, pl.ds(s * RPT, RPT)])


@jax.jit
def _aggregate(y_p, edge_p, zeros_p):
    return pl.kernel(
        _agg_body,
        out_type=jax.ShapeDtypeStruct((NCU, NP, D), jnp.float32),
        mesh=_mesh(),
        compiler_params=pltpu.CompilerParams(needs_layout_passes=False),
        scratch_types=[
            pltpu.VMEM((2, KP, CE), jnp.int32),
            pltpu.VMEM((2, CE, D), jnp.float32),
            pltpu.VMEM_SHARED((NP, D), jnp.float32),
            pltpu.SemaphoreType.DMA((4,)),
        ],
    )(y_p, edge_p, zeros_p)


def _norms(degb):
    deg_src = jnp.sum(degb[:NW], axis=0)
    deg_dst = jnp.sum(degb[NW:], axis=0)
    n_src = lax.rsqrt(jnp.clip(deg_src, 1.0, None))
    n_dst = lax.rsqrt(jnp.clip(deg_dst, 1.0, None))
    return n_src, n_dst


def _tca_body(xb, w1, degb, yb):
    n_src, _ = _norms(degb)
    yb[...] = jnp.dot(xb[...], w1[...],
                      preferred_element_type=jnp.float32) * n_src[:, None]


@jax.jit
def _tc_a(x_p, W1, degp):
    grid = NP // BM
    return pl.pallas_call(
        _tca_body,
        grid=(grid,),
        in_specs=[
            pl.BlockSpec((BM, D), lambda i: (i, 0)),
            pl.BlockSpec((D, D), lambda i: (0, 0)),
            pl.BlockSpec((2 * NW, BM), lambda i: (0, i)),
        ],
        out_specs=pl.BlockSpec((BM, D), lambda i: (i, 0)),
        out_shape=jax.ShapeDtypeStruct((NP, D), jnp.float32),
    )(x_p, W1, degp)


def _tcb_body(aggb, w2, b1b, degb, yb):
    n_src, n_dst = _norms(degb)
    row = pl.program_id(0) * BM + lax.broadcasted_iota(jnp.int32, (BM,), 0)
    n_src = jnp.where(row < N, n_src, 0.0)
    agg = sum(aggb[i] for i in range(1, NCU)) + aggb[0]
    h = jax.nn.relu(agg * n_dst[:, None] + b1b[...])
    yb[...] = jnp.dot(h, w2[...],
                      preferred_element_type=jnp.float32) * n_src[:, None]


@jax.jit
def _tc_b(agg, W2, b1, degp):
    grid = NP // BM
    return pl.pallas_call(
        _tcb_body,
        grid=(grid,),
        in_specs=[
            pl.BlockSpec((NCU, BM, D), lambda i: (0, i, 0)),
            pl.BlockSpec((D, D), lambda i: (0, 0)),
            pl.BlockSpec((1, D), lambda i: (0, 0)),
            pl.BlockSpec((2 * NW, BM), lambda i: (0, i)),
        ],
        out_specs=pl.BlockSpec((BM, D), lambda i: (i, 0)),
        out_shape=jax.ShapeDtypeStruct((NP, D), jnp.float32),
    )(agg, W2, b1.reshape(1, D), degp)


def _tcc_body(aggb, wc, b2b, bcb, degb, ob):
    _, n_dst = _norms(degb)
    agg = sum(aggb[i] for i in range(1, NCU)) + aggb[0]
    h = jax.nn.relu(agg * n_dst[:, None] + b2b[...])
    ob[...] = jnp.dot(h, wc[...], preferred_element_type=jnp.float32) + bcb[...]


@jax.jit
def _tc_c(agg, Wc_p, b2, bc_p, degp):
    grid = NP // BM
    return pl.pallas_call(
        _tcc_body,
        grid=(grid,),
        in_specs=[
            pl.BlockSpec((NCU, BM, D), lambda i: (0, i, 0)),
            pl.BlockSpec((D, D), lambda i: (0, 0)),
            pl.BlockSpec((1, D), lambda i: (0, 0)),
            pl.BlockSpec((1, D), lambda i: (0, 0)),
            pl.BlockSpec((2 * NW, BM), lambda i: (0, i)),
        ],
        out_specs=pl.BlockSpec((BM, D), lambda i: (i, 0)),
        out_shape=jax.ShapeDtypeStruct((NP, D), jnp.float32),
    )(agg, Wc_p, b2.reshape(1, D), bc_p, degp)


def kernel(x, edge_index, W1, b1, W2, b2, Wc, bc):
    src = edge_index[0].astype(jnp.int32)
    dst = edge_index[1].astype(jnp.int32)
    padfill = jnp.full((E_PAD - E,), N, jnp.int32)
    src_p = jnp.concatenate([src, padfill])
    dst_p = jnp.concatenate([dst, padfill])

    e0 = NS * K0 * CE

    def _tile_view(a):
        a0 = a[:e0].reshape(NS, NPH0, KP, CE)
        a1 = a[e0:].reshape(NS, NPH1, KP, CE)
        a1 = jnp.pad(a1, ((0, 0), (0, NPH0 - NPH1), (0, 0), (0, 0)),
                     constant_values=N)
        return jnp.concatenate([a1, a0], axis=0)

    edge_p = jnp.stack([_tile_view(src_p), _tile_view(dst_p)], axis=2)
    zeros_p = jnp.zeros((NP, D), jnp.float32)
    src_d = src_p.reshape(NW, E_PAD // NW)
    dst_d = dst_p.reshape(NW, E_PAD // NW)

    x_p = jnp.pad(x, ((0, NP - N), (0, 0)))
    Wc_p = jnp.pad(Wc, ((0, 0), (0, D - D_OUT)))
    bc_p = jnp.pad(bc, ((0, D - D_OUT),)).reshape(1, D)

    degp = _degrees(src_d, dst_d).reshape(2 * NW, NP)

    y1 = _tc_a(x_p, W1, degp)
    agg1 = _aggregate(y1, edge_p, zeros_p)
    y2 = _tc_b(agg1, W2, b1, degp)
    agg2 = _aggregate(y2, edge_p, zeros_p)
    out = _tc_c(agg2, Wc_p, b2, bc_p, degp)
    return out[:N, :D_OUT]


# speedup vs baseline: 1.3846x; 1.0003x over previous
import jax
import jax.numpy as jnp
from jax import lax
from jax.experimental import pallas as pl
from jax.experimental.pallas import tpu as pltpu
from jax.experimental.pallas import tpu_sc as plsc

N = 10000
E = 320000
D = 128
DH = D // 2
D_OUT = 40

NC = 2
NCU = 2
NS = 16
NW = NCU * NS

CE = 128
KP = 40
NPH0 = 2
NPH1 = 2
K0 = NPH0 * KP
K1 = NPH1 * KP
E_PAD = NS * (K0 + K1) * CE
NP = 10240
RPT = NP // NS

BM = 256


def _mesh():
    return plsc.VectorSubcoreMesh(core_axis_name="c", subcore_axis_name="s", num_cores=NCU)


def _degree_body(src_hbm, dst_hbm, degp_hbm, sidx, didx, hist_s, hist_d):
    c = lax.axis_index("c")
    s = lax.axis_index("s")
    wid = c * NS + s

    pltpu.sync_copy(src_hbm.at[wid], sidx)
    pltpu.sync_copy(dst_hbm.at[wid], didx)

    zero16 = jnp.zeros((16,), jnp.float32)

    def zinit(i, _):
        hist_s[pl.ds(i * 16, 16)] = zero16
        hist_d[pl.ds(i * 16, 16)] = zero16
        return 0

    lax.fori_loop(0, NP // 16, zinit, 0)

    ones16 = jnp.ones((16,), jnp.float32)

    def body(i, _):
        sv = sidx[pl.ds(i * 16, 16)]
        dv = didx[pl.ds(i * 16, 16)]
        plsc.addupdate_scatter(hist_s, [sv], ones16)
        plsc.addupdate_scatter(hist_d, [dv], ones16)
        return 0

    lax.fori_loop(0, (E_PAD // NW) // 16, body, 0)

    pltpu.sync_copy(hist_s, degp_hbm.at[0, wid])
    pltpu.sync_copy(hist_d, degp_hbm.at[1, wid])


@jax.jit
def _degrees(src_p, dst_p):
    return pl.kernel(
        _degree_body,
        out_type=jax.ShapeDtypeStruct((2, NW, NP), jnp.float32),
        mesh=_mesh(),
        compiler_params=pltpu.CompilerParams(needs_layout_passes=False),
        scratch_types=[
            pltpu.VMEM((E_PAD // NW,), jnp.int32),
            pltpu.VMEM((E_PAD // NW,), jnp.int32),
            pltpu.VMEM((NP,), jnp.float32),
            pltpu.VMEM((NP,), jnp.float32),
        ],
    )(src_p, dst_p)


def _agg_body(y_hbm, edge_hbm, zeros_hbm, out_hbm, eidx, rr, acc, gsem):
    c = lax.axis_index("c")
    s = lax.axis_index("s")
    wid = c * NS + s

    pltpu.sync_copy(zeros_hbm.at[pl.ds(s * RPT, RPT)],
                    acc.at[pl.ds(s * RPT, RPT)])

    plsc.subcore_barrier()

    nph = NPH0

    def phase_body(ph, _):
        pltpu.sync_copy(edge_hbm.at[wid, ph], eidx)

        for b in range(2):
            pltpu.async_copy(y_hbm.at[eidx.at[0, b]], rr.at[b], gsem.at[b])

        def round_body(r, _):
            for b in range(2):
                j = 2 * r + b
                pltpu.make_async_copy(
                    y_hbm.at[eidx.at[0, j]], rr.at[b], gsem.at[b]).wait()
                pltpu.sync_copy(rr.at[b], acc.at[eidx.at[1, j]], add=True)
                pltpu.async_copy(y_hbm.at[eidx.at[0, j + 2]], rr.at[b],
                                 gsem.at[b])
            return 0

        lax.fori_loop(0, KP // 2 - 1, round_body, 0)
        for b in range(2):
            j = KP - 2 + b
            pltpu.make_async_copy(
                y_hbm.at[eidx.at[0, j]], rr.at[b], gsem.at[b]).wait()
            pltpu.sync_copy(rr.at[b], acc.at[eidx.at[1, j]], add=True)
        return 0

    lax.fori_loop(0, nph, phase_body, 0)

    plsc.subcore_barrier()

    pltpu.sync_copy(acc.at[pl.ds(s * RPT, RPT)],
                    out_hbm.at[c, pl.ds(s * RPT, RPT)])


@jax.jit
def _aggregate(y_p, edge_p, zeros_p):
    return pl.kernel(
        _agg_body,
        out_type=jax.ShapeDtypeStruct((NCU, NP, D), jnp.float32),
        mesh=_mesh(),
        compiler_params=pltpu.CompilerParams(needs_layout_passes=False),
        scratch_types=[
            pltpu.VMEM((2, KP, CE), jnp.int32),
            pltpu.VMEM((2, CE, D), jnp.float32),
            pltpu.VMEM_SHARED((NP, D), jnp.float32),
            pltpu.SemaphoreType.DMA((2,)),
        ],
    )(y_p, edge_p, zeros_p)


def _norms(degb):
    deg_src = jnp.sum(degb[:NW], axis=0)
    deg_dst = jnp.sum(degb[NW:], axis=0)
    n_src = lax.rsqrt(jnp.clip(deg_src, 1.0, None))
    n_dst = lax.rsqrt(jnp.clip(deg_dst, 1.0, None))
    return n_src, n_dst


def _tca_body(xb, w1, degb, yb):
    n_src, _ = _norms(degb)
    yb[...] = jnp.dot(xb[...], w1[...],
                      preferred_element_type=jnp.float32) * n_src[:, None]


@jax.jit
def _tc_a(x_p, W1, degp):
    grid = NP // BM
    return pl.pallas_call(
        _tca_body,
        grid=(grid,),
        in_specs=[
            pl.BlockSpec((BM, D), lambda i: (i, 0)),
            pl.BlockSpec((D, D), lambda i: (0, 0)),
            pl.BlockSpec((2 * NW, BM), lambda i: (0, i)),
        ],
        out_specs=pl.BlockSpec((BM, D), lambda i: (i, 0)),
        out_shape=jax.ShapeDtypeStruct((NP, D), jnp.float32),
    )(x_p, W1, degp)


def _tcb_body(aggb, w2, b1b, degb, yb):
    n_src, n_dst = _norms(degb)
    row = pl.program_id(0) * BM + lax.broadcasted_iota(jnp.int32, (BM,), 0)
    n_src = jnp.where(row < N, n_src, 0.0)
    agg = sum(aggb[i] for i in range(1, NCU)) + aggb[0]
    h = jax.nn.relu(agg * n_dst[:, None] + b1b[...])
    yb[...] = jnp.dot(h, w2[...],
                      preferred_element_type=jnp.float32) * n_src[:, None]


@jax.jit
def _tc_b(agg, W2, b1, degp):
    grid = NP // BM
    return pl.pallas_call(
        _tcb_body,
        grid=(grid,),
        in_specs=[
            pl.BlockSpec((NCU, BM, D), lambda i: (0, i, 0)),
            pl.BlockSpec((D, D), lambda i: (0, 0)),
            pl.BlockSpec((1, D), lambda i: (0, 0)),
            pl.BlockSpec((2 * NW, BM), lambda i: (0, i)),
        ],
        out_specs=pl.BlockSpec((BM, D), lambda i: (i, 0)),
        out_shape=jax.ShapeDtypeStruct((NP, D), jnp.float32),
    )(agg, W2, b1.reshape(1, D), degp)


def _tcc_body(aggb, wc, b2b, bcb, degb, ob):
    _, n_dst = _norms(degb)
    agg = sum(aggb[i] for i in range(1, NCU)) + aggb[0]
    h = jax.nn.relu(agg * n_dst[:, None] + b2b[...])
    ob[...] = jnp.dot(h, wc[...], preferred_element_type=jnp.float32) + bcb[...]


@jax.jit
def _tc_c(agg, Wc_p, b2, bc_p, degp):
    grid = NP // BM
    return pl.pallas_call(
        _tcc_body,
        grid=(grid,),
        in_specs=[
            pl.BlockSpec((NCU, BM, D), lambda i: (0, i, 0)),
            pl.BlockSpec((D, D), lambda i: (0, 0)),
            pl.BlockSpec((1, D), lambda i: (0, 0)),
            pl.BlockSpec((1, D), lambda i: (0, 0)),
            pl.BlockSpec((2 * NW, BM), lambda i: (0, i)),
        ],
        out_specs=pl.BlockSpec((BM, D), lambda i: (i, 0)),
        out_shape=jax.ShapeDtypeStruct((NP, D), jnp.float32),
    )(agg, Wc_p, b2.reshape(1, D), bc_p, degp)


def kernel(x, edge_index, W1, b1, W2, b2, Wc, bc):
    src = edge_index[0].astype(jnp.int32)
    dst = edge_index[1].astype(jnp.int32)
    padfill = jnp.full((E_PAD - E,), N, jnp.int32)
    src_p = jnp.concatenate([src, padfill])
    dst_p = jnp.concatenate([dst, padfill])

    e0 = NS * K0 * CE

    def _tile_view(a):
        a0 = a[:e0].reshape(NS, NPH0, KP, CE)
        a1 = a[e0:].reshape(NS, NPH1, KP, CE)
        a1 = jnp.pad(a1, ((0, 0), (0, NPH0 - NPH1), (0, 0), (0, 0)),
                     constant_values=N)
        return jnp.concatenate([a1, a0], axis=0)

    edge_p = jnp.stack([_tile_view(src_p), _tile_view(dst_p)], axis=2)
    zeros_p = jnp.zeros((NP, D), jnp.float32)
    src_d = src_p.reshape(NW, E_PAD // NW)
    dst_d = dst_p.reshape(NW, E_PAD // NW)

    x_p = jnp.pad(x, ((0, NP - N), (0, 0)))
    Wc_p = jnp.pad(Wc, ((0, 0), (0, D - D_OUT)))
    bc_p = jnp.pad(bc, ((0, D - D_OUT),)).reshape(1, D)

    degp = _degrees(src_d, dst_d).reshape(2 * NW, NP)

    y1 = _tc_a(x_p, W1, degp)
    agg1 = _aggregate(y1, edge_p, zeros_p)
    y2 = _tc_b(agg1, W2, b1, degp)
    agg2 = _aggregate(y2, edge_p, zeros_p)
    out = _tc_c(agg2, Wc_p, b2, bc_p, degp)
    return out[:N, :D_OUT]
